# baseline (speedup 1.0000x reference)
import jax
import jax.numpy as jnp
from jax import lax
from jax.experimental import pallas as pl
from jax.experimental.pallas import tpu as pltpu
from jax.experimental.pallas import tpu_sc as plsc

NC = 2
NS = 16
NW = NC * NS
L = 16


def _sc_body(fr, smap, opsr, invr, latr, nopr, out,
             rawv, smv, opsv, invv, latv, nopv,
             f0v, f1v, f2v, accx, accy, accz):
    NA = f0v.shape[0] // L
    NOP = smv.shape[0] // (NA * L)

    wid = lax.axis_index("c") * NS + lax.axis_index("s")

    pltpu.sync_copy(fr.at[pl.ds(wid * NA * L * 3, NA * L * 3)], rawv)
    pltpu.sync_copy(smap.at[pl.ds(wid * NOP * NA * L, NOP * NA * L)], smv)
    pltpu.sync_copy(opsr.at[pl.ds(wid * NOP * 16 * L, NOP * 16 * L)], opsv)
    pltpu.sync_copy(invr.at[pl.ds(wid * 9 * L, 9 * L)], invv)
    pltpu.sync_copy(latr.at[pl.ds(wid * 9 * L, 9 * L)], latv)
    pltpu.sync_copy(nopr.at[pl.ds(wid * L, L)], nopv)

    lane = lax.iota(jnp.int32, L)
    lane768 = lane * (NA * 3)
    laneNA = lane * NA
    lane9 = lane * 9
    lane128 = lane * (NOP * 16)
    lane2048 = lane * (NOP * NA)

    zero = jnp.zeros((L,), jnp.float32)

    @pl.loop(0, NA, unroll=8)
    def _prep(a):
        al = a * L
        accx[pl.ds(al, L)] = zero
        accy[pl.ds(al, L)] = zero
        accz[pl.ds(al, L)] = zero
        base = lane768 + (3 * a)
        f0v[pl.ds(al, L)] = plsc.load_gather(rawv, [base])
        f1v[pl.ds(al, L)] = plsc.load_gather(rawv, [base + 1])
        f2v[pl.ds(al, L)] = plsc.load_gather(rawv, [base + 2])

    scale = 1.0 / nopv[pl.ds(0, L)].astype(jnp.float32)
    inv_s = [[plsc.load_gather(invv, [lane9 + (j * 3 + l)]) * scale
              for l in range(3)] for j in range(3)]
    lat_v = [[plsc.load_gather(latv, [lane9 + (k * 3 + i)])
              for i in range(3)] for k in range(3)]

    for o in range(NOP):
        r_v = [[plsc.load_gather(opsv, [lane128 + (o * 16 + k * 4 + l)])
                for l in range(3)] for k in range(3)]
        m = [[None] * 3 for _ in range(3)]
        for j in range(3):
            t1 = [r_v[k][0] * inv_s[j][0] + r_v[k][1] * inv_s[j][1]
                  + r_v[k][2] * inv_s[j][2] for k in range(3)]
            for i in range(3):
                m[j][i] = (t1[0] * lat_v[0][i] + t1[1] * lat_v[1][i]
                           + t1[2] * lat_v[2][i])

        sm_o = o * NA

        @pl.loop(0, NA, unroll=4)
        def _atoms(a):
            al = a * L
            f0 = f0v[pl.ds(al, L)]
            f1 = f1v[pl.ds(al, L)]
            f2 = f2v[pl.ds(al, L)]
            idx = plsc.load_gather(smv, [lane2048 + (sm_o + a)]) + laneNA
            gx = f0 * m[0][0] + f1 * m[1][0] + f2 * m[2][0]
            gy = f0 * m[0][1] + f1 * m[1][1] + f2 * m[2][1]
            gz = f0 * m[0][2] + f1 * m[1][2] + f2 * m[2][2]
            plsc.addupdate_scatter(accx, [idx], gx)
            plsc.addupdate_scatter(accy, [idx], gy)
            plsc.addupdate_scatter(accz, [idx], gz)

    iota3 = lane * 3

    @pl.loop(0, NA, unroll=8)
    def _epi(a):
        al = a * L
        base = iota3 + (a * L * 3)
        plsc.store_scatter(rawv, [base], accx[pl.ds(al, L)])
        plsc.store_scatter(rawv, [base + 1], accy[pl.ds(al, L)])
        plsc.store_scatter(rawv, [base + 2], accz[pl.ds(al, L)])

    pltpu.sync_copy(rawv, out.at[pl.ds(wid * NA * L * 3, NA * L * 3)])


def kernel(lattices, inv_lattices, forces, batch, num_atoms, general_ops,
           symm_map, num_general_ops):
    B = lattices.shape[0]
    NOP = symm_map.shape[1]
    NA = symm_map.shape[2]
    N = forces.shape[0]

    mesh = plsc.VectorSubcoreMesh(core_axis_name="c", subcore_axis_name="s",
                                  num_cores=NC, num_subcores=NS)
    run = pl.kernel(
        _sc_body,
        out_type=jax.ShapeDtypeStruct((N * 3,), jnp.float32),
        mesh=mesh,
        compiler_params=pltpu.CompilerParams(needs_layout_passes=False),
        scratch_types=[
            pltpu.VMEM((NA * L * 3,), jnp.float32),
            pltpu.VMEM((NOP * NA * L,), jnp.int32),
            pltpu.VMEM((NOP * 16 * L,), jnp.float32),
            pltpu.VMEM((9 * L,), jnp.float32),
            pltpu.VMEM((9 * L,), jnp.float32),
            pltpu.VMEM((L,), jnp.int32),
            pltpu.VMEM((NA * L,), jnp.float32),
            pltpu.VMEM((NA * L,), jnp.float32),
            pltpu.VMEM((NA * L,), jnp.float32),
            pltpu.VMEM((NA * L,), jnp.float32),
            pltpu.VMEM((NA * L,), jnp.float32),
            pltpu.VMEM((NA * L,), jnp.float32),
        ],
    )
    out = run(forces.reshape(-1), symm_map.reshape(-1),
              general_ops.reshape(-1), inv_lattices.reshape(-1),
              lattices.reshape(-1), num_general_ops)
    return out.reshape(N, 3)


# speedup vs baseline: 5.0310x; 5.0310x over previous
import jax
import jax.numpy as jnp
from jax import lax
from jax.experimental import pallas as pl
from jax.experimental.pallas import tpu as pltpu
from jax.experimental.pallas import tpu_sc as plsc

NC = 2
NS = 16
NW = NC * NS
L = 16


def _sc_body(fr, smap, opsr, invr, latr, nopr, out,
             rawv, smv, opsv, invv, latv, nopv, accx, accy, accz, sem):
    NA = 256
    NOP = 8

    wid = lax.axis_index("c") * NS + lax.axis_index("s")
    blk = wid // 8
    boff = (wid % 8) * 16

    d = []
    d.append(pltpu.async_copy(fr.at[pl.ds(wid * 16384, 16384)], rawv, sem))
    d.append(pltpu.async_copy(smap.at[pl.ds(wid * 32768, 32768)], smv, sem))
    for k in range(3):
        d.append(pltpu.async_copy(
            opsr.at[pl.ds(k * 16384 + wid * 512, 512)],
            opsv.at[pl.ds(k * 512, 512)], sem))
    for j in range(3):
        d.append(pltpu.async_copy(
            invr.at[pl.ds(j * 2048 + blk * 512, 512)],
            invv.at[pl.ds(j * 512, 512)], sem))
        d.append(pltpu.async_copy(
            latr.at[pl.ds(j * 2048 + blk * 512, 512)],
            latv.at[pl.ds(j * 512, 512)], sem))
    d.append(pltpu.async_copy(nopr.at[pl.ds(wid * L, L)], nopv, sem))

    lane = lax.iota(jnp.int32, L)
    lane8 = lane * 8
    lane1024 = lane * 1024
    lane2048 = lane * 2048
    zero = jnp.zeros((L,), jnp.float32)

    @pl.loop(0, NA * L, step=L, unroll=8)
    def _zero(i):
        accx[pl.ds(i, L)] = zero
        accy[pl.ds(i, L)] = zero
        accz[pl.ds(i, L)] = zero

    for de in d:
        de.wait()

    scale = 1.0 / nopv[pl.ds(0, L)].astype(jnp.float32)
    lane_b = lane + boff
    inv_s = [[plsc.load_gather(invv, [lane_b + (j * 512 + l * 128)]) * scale
              for l in range(3)] for j in range(3)]
    lat_v = [[plsc.load_gather(latv, [lane_b + (k * 512 + i * 128)])
              for i in range(3)] for k in range(3)]

    for o in range(NOP):
        r_v = [[plsc.load_gather(opsv, [lane8 + (k * 512 + l * 128 + o)])
                for l in range(3)] for k in range(3)]
        m = [[None] * 3 for _ in range(3)]
        for j in range(3):
            t1 = [r_v[k][0] * inv_s[j][0] + r_v[k][1] * inv_s[j][1]
                  + r_v[k][2] * inv_s[j][2] for k in range(3)]
            for i in range(3):
                m[j][i] = (t1[0] * lat_v[0][i] + t1[1] * lat_v[1][i]
                           + t1[2] * lat_v[2][i])

        for b0 in range(2):
            f_ref = [rawv.at[pl.ds(b0 * 512 + c * 128, 15616)]
                     for c in range(3)]
            s_ref = smv.at[pl.ds(b0 * 1024 + o * 128, 30848)]

            @pl.loop(0, 128, unroll=8)
            def _hot(a0):
                alow = (lane + a0) & 127
                fidx = lane1024 + alow
                f0 = plsc.load_gather(f_ref[0], [fidx])
                f1 = plsc.load_gather(f_ref[1], [fidx])
                f2 = plsc.load_gather(f_ref[2], [fidx])
                mv = plsc.load_gather(s_ref, [lane2048 + alow])
                widx = (mv << 4) + lane
                gx = f0 * m[0][0] + f1 * m[1][0] + f2 * m[2][0]
                gy = f0 * m[0][1] + f1 * m[1][1] + f2 * m[2][1]
                gz = f0 * m[0][2] + f1 * m[1][2] + f2 * m[2][2]
                plsc.addupdate_scatter(accx, [widx], gx)
                plsc.addupdate_scatter(accy, [widx], gy)
                plsc.addupdate_scatter(accz, [widx], gz)

    for b0 in range(2):
        acc_ref = [accx.at[pl.ds(b0 * 2048, 2048)],
                   accy.at[pl.ds(b0 * 2048, 2048)],
                   accz.at[pl.ds(b0 * 2048, 2048)]]
        w_ref = [rawv.at[pl.ds(b0 * 512 + c * 128, 15616)] for c in range(3)]

        @pl.loop(0, 128, unroll=8)
        def _epi(a0):
            alow = (lane + a0) & 127
            ridx = (alow << 4) + lane
            woff = lane1024 + alow
            plsc.store_scatter(w_ref[0], [woff],
                               plsc.load_gather(acc_ref[0], [ridx]))
            plsc.store_scatter(w_ref[1], [woff],
                               plsc.load_gather(acc_ref[1], [ridx]))
            plsc.store_scatter(w_ref[2], [woff],
                               plsc.load_gather(acc_ref[2], [ridx]))

    pltpu.sync_copy(rawv, out.at[pl.ds(wid * 16384, 16384)])


def kernel(lattices, inv_lattices, forces, batch, num_atoms, general_ops,
           symm_map, num_general_ops):
    B = lattices.shape[0]
    NOP = symm_map.shape[1]
    NA = symm_map.shape[2]
    N = forces.shape[0]

    f_t = (jnp.pad(forces, ((0, 0), (0, 1)))
           .reshape(N // 128, 128, 4).transpose(0, 2, 1).reshape(-1))
    sm_t = (symm_map.reshape(B, NOP, NA // 128, 128)
            .transpose(0, 2, 1, 3).reshape(-1))
    ops_t = (general_ops.reshape(B * NOP // 128, 128, 4, 4)
             .transpose(2, 0, 3, 1).reshape(-1))
    inv_t = (jnp.pad(inv_lattices, ((0, 0), (0, 0), (0, 1)))
             .reshape(B // 128, 128, 3, 4).transpose(2, 0, 3, 1).reshape(-1))
    lat_t = (jnp.pad(lattices, ((0, 0), (0, 0), (0, 1)))
             .reshape(B // 128, 128, 3, 4).transpose(2, 0, 3, 1).reshape(-1))

    mesh = plsc.VectorSubcoreMesh(core_axis_name="c", subcore_axis_name="s",
                                  num_cores=NC, num_subcores=NS)
    run = pl.kernel(
        _sc_body,
        out_type=jax.ShapeDtypeStruct((N * 4,), jnp.float32),
        mesh=mesh,
        compiler_params=pltpu.CompilerParams(needs_layout_passes=False),
        scratch_types=[
            pltpu.VMEM((16384,), jnp.float32),
            pltpu.VMEM((32768,), jnp.int32),
            pltpu.VMEM((1536,), jnp.float32),
            pltpu.VMEM((1536,), jnp.float32),
            pltpu.VMEM((1536,), jnp.float32),
            pltpu.VMEM((L,), jnp.int32),
            pltpu.VMEM((NA * L,), jnp.float32),
            pltpu.VMEM((NA * L,), jnp.float32),
            pltpu.VMEM((NA * L,), jnp.float32),
            pltpu.SemaphoreType.DMA,
        ],
    )
    out = run(f_t, sm_t, ops_t, inv_t, lat_t, num_general_ops)
    return (out.reshape(N // 128, 4, 128).transpose(0, 2, 1)
            .reshape(N, 4)[:, :3])


# speedup vs baseline: 5.0866x; 1.0110x over previous
import jax
import jax.numpy as jnp
from jax import lax
from jax.experimental import pallas as pl
from jax.experimental.pallas import tpu as pltpu
from jax.experimental.pallas import tpu_sc as plsc

NC = 2
NS = 16
NW = NC * NS
L = 16


def _sc_body(fr, smap, opsr, invr, latr, nopr, out,
             rawv, smv, opsv, invv, latv, nopv, mbuf,
             accx, accy, accz, semb, sems):
    NA = 256
    NOP = 8

    wid = lax.axis_index("c") * NS + lax.axis_index("s")
    blk = wid // 8
    boff = (wid % 8) * 16

    db = []
    db.append(pltpu.async_copy(fr.at[pl.ds(wid * 16384, 16384)], rawv, semb))
    db.append(pltpu.async_copy(smap.at[pl.ds(wid * 32768, 32768)], smv, semb))
    ds = []
    for k in range(3):
        ds.append(pltpu.async_copy(
            opsr.at[pl.ds(k * 16384 + wid * 512, 512)],
            opsv.at[pl.ds(k * 512, 512)], sems))
    for j in range(3):
        ds.append(pltpu.async_copy(
            invr.at[pl.ds(j * 2048 + blk * 512, 512)],
            invv.at[pl.ds(j * 512, 512)], sems))
        ds.append(pltpu.async_copy(
            latr.at[pl.ds(j * 2048 + blk * 512, 512)],
            latv.at[pl.ds(j * 512, 512)], sems))
    ds.append(pltpu.async_copy(nopr.at[pl.ds(wid * L, L)], nopv, sems))

    lane = lax.iota(jnp.int32, L)
    lane8 = lane * 8
    lane1024 = lane * 1024
    lane2048 = lane * 2048
    zero = jnp.zeros((L,), jnp.float32)

    @pl.loop(0, NA * L, step=L, unroll=8)
    def _zero(i):
        accx[pl.ds(i, L)] = zero
        accy[pl.ds(i, L)] = zero
        accz[pl.ds(i, L)] = zero

    for de in ds:
        de.wait()

    scale = 1.0 / nopv[pl.ds(0, L)].astype(jnp.float32)
    lane_b = lane + boff
    inv_s = [[plsc.load_gather(invv, [lane_b + (j * 512 + l * 128)]) * scale
              for l in range(3)] for j in range(3)]
    lat_v = [[plsc.load_gather(latv, [lane_b + (k * 512 + i * 128)])
              for i in range(3)] for k in range(3)]

    for o in range(NOP):
        r_v = [[plsc.load_gather(opsv, [lane8 + (k * 512 + l * 128 + o)])
                for l in range(3)] for k in range(3)]
        for j in range(3):
            t1 = [r_v[k][0] * inv_s[j][0] + r_v[k][1] * inv_s[j][1]
                  + r_v[k][2] * inv_s[j][2] for k in range(3)]
            for i in range(3):
                mbuf[pl.ds((o * 9 + j * 3 + i) * L, L)] = (
                    t1[0] * lat_v[0][i] + t1[1] * lat_v[1][i]
                    + t1[2] * lat_v[2][i])

    for de in db:
        de.wait()

    for o in range(NOP):
        m = [[mbuf[pl.ds((o * 9 + j * 3 + i) * L, L)] for i in range(3)]
             for j in range(3)]

        for b0 in range(2):
            f_ref = [rawv.at[pl.ds(b0 * 512 + c * 128, 15616)]
                     for c in range(3)]
            s_ref = smv.at[pl.ds(b0 * 1024 + o * 128, 30848)]

            @pl.loop(0, 128, unroll=8)
            def _hot(a0):
                alow = lane ^ a0
                fidx = lane1024 + alow
                f0 = plsc.load_gather(f_ref[0], [fidx])
                f1 = plsc.load_gather(f_ref[1], [fidx])
                f2 = plsc.load_gather(f_ref[2], [fidx])
                mv = plsc.load_gather(s_ref, [lane2048 + alow])
                widx = (mv << 4) + lane
                gx = f0 * m[0][0] + f1 * m[1][0] + f2 * m[2][0]
                gy = f0 * m[0][1] + f1 * m[1][1] + f2 * m[2][1]
                gz = f0 * m[0][2] + f1 * m[1][2] + f2 * m[2][2]
                plsc.addupdate_scatter(accx, [widx], gx)
                plsc.addupdate_scatter(accy, [widx], gy)
                plsc.addupdate_scatter(accz, [widx], gz)

    for b0 in range(2):
        acc_ref = [accx.at[pl.ds(b0 * 2048, 2048)],
                   accy.at[pl.ds(b0 * 2048, 2048)],
                   accz.at[pl.ds(b0 * 2048, 2048)]]
        w_ref = [rawv.at[pl.ds(b0 * 512 + c * 128, 15616)] for c in range(3)]

        @pl.loop(0, 128, unroll=8)
        def _epi(a0):
            alow = lane ^ a0
            ridx = (alow << 4) + lane
            woff = lane1024 + alow
            plsc.store_scatter(w_ref[0], [woff],
                               plsc.load_gather(acc_ref[0], [ridx]))
            plsc.store_scatter(w_ref[1], [woff],
                               plsc.load_gather(acc_ref[1], [ridx]))
            plsc.store_scatter(w_ref[2], [woff],
                               plsc.load_gather(acc_ref[2], [ridx]))

    pltpu.sync_copy(rawv, out.at[pl.ds(wid * 16384, 16384)])


def kernel(lattices, inv_lattices, forces, batch, num_atoms, general_ops,
           symm_map, num_general_ops):
    B = lattices.shape[0]
    NOP = symm_map.shape[1]
    NA = symm_map.shape[2]
    N = forces.shape[0]

    f_t = (jnp.pad(forces, ((0, 0), (0, 1)))
           .reshape(N // 128, 128, 4).transpose(0, 2, 1).reshape(-1))
    sm_t = (symm_map.reshape(B, NOP, NA // 128, 128)
            .transpose(0, 2, 1, 3).reshape(-1))
    ops_t = (general_ops.reshape(B * NOP // 128, 128, 4, 4)
             .transpose(2, 0, 3, 1).reshape(-1))
    inv_t = (jnp.pad(inv_lattices, ((0, 0), (0, 0), (0, 1)))
             .reshape(B // 128, 128, 3, 4).transpose(2, 0, 3, 1).reshape(-1))
    lat_t = (jnp.pad(lattices, ((0, 0), (0, 0), (0, 1)))
             .reshape(B // 128, 128, 3, 4).transpose(2, 0, 3, 1).reshape(-1))

    mesh = plsc.VectorSubcoreMesh(core_axis_name="c", subcore_axis_name="s",
                                  num_cores=NC, num_subcores=NS)
    run = pl.kernel(
        _sc_body,
        out_type=jax.ShapeDtypeStruct((N * 4,), jnp.float32),
        mesh=mesh,
        compiler_params=pltpu.CompilerParams(needs_layout_passes=False),
        scratch_types=[
            pltpu.VMEM((16384,), jnp.float32),
            pltpu.VMEM((32768,), jnp.int32),
            pltpu.VMEM((1536,), jnp.float32),
            pltpu.VMEM((1536,), jnp.float32),
            pltpu.VMEM((1536,), jnp.float32),
            pltpu.VMEM((L,), jnp.int32),
            pltpu.VMEM((NOP * 9 * L,), jnp.float32),
            pltpu.VMEM((NA * L,), jnp.float32),
            pltpu.VMEM((NA * L,), jnp.float32),
            pltpu.VMEM((NA * L,), jnp.float32),
            pltpu.SemaphoreType.DMA,
            pltpu.SemaphoreType.DMA,
        ],
    )
    out = run(f_t, sm_t, ops_t, inv_t, lat_t, num_general_ops)
    return (out.reshape(N // 128, 4, 128).transpose(0, 2, 1)
            .reshape(N, 4)[:, :3])


# speedup vs baseline: 5.1137x; 1.0053x over previous
import jax
import jax.numpy as jnp
from jax import lax
from jax.experimental import pallas as pl
from jax.experimental.pallas import tpu as pltpu
from jax.experimental.pallas import tpu_sc as plsc

NC = 2
NS = 16
NW = NC * NS
L = 16


def _sc_body(fr, smap, opsr, invr, latr, nopr, out,
             rawv, smv, opsv, invv, latv, nopv, mbuf,
             accx, accy, accz, semb, sems):
    NA = 256
    NOP = 8

    wid = lax.axis_index("c") * NS + lax.axis_index("s")
    blk = wid // 8
    boff = (wid % 8) * 16

    db = []
    db.append(pltpu.async_copy(fr.at[pl.ds(wid * 16384, 16384)], rawv, semb))
    db.append(pltpu.async_copy(smap.at[pl.ds(wid * 32768, 32768)], smv, semb))
    ds = []
    for k in range(3):
        ds.append(pltpu.async_copy(
            opsr.at[pl.ds(k * 16384 + wid * 512, 512)],
            opsv.at[pl.ds(k * 512, 512)], sems))
    for j in range(3):
        ds.append(pltpu.async_copy(
            invr.at[pl.ds(j * 2048 + blk * 512, 512)],
            invv.at[pl.ds(j * 512, 512)], sems))
        ds.append(pltpu.async_copy(
            latr.at[pl.ds(j * 2048 + blk * 512, 512)],
            latv.at[pl.ds(j * 512, 512)], sems))
    ds.append(pltpu.async_copy(nopr.at[pl.ds(wid * L, L)], nopv, sems))

    lane = lax.iota(jnp.int32, L)
    lane8 = lane * 8
    lane1024 = lane * 1024
    lane2048 = lane * 2048
    zero = jnp.zeros((L,), jnp.float32)

    @pl.loop(0, NA * L, step=L, unroll=8)
    def _zero(i):
        accx[pl.ds(i, L)] = zero
        accy[pl.ds(i, L)] = zero
        accz[pl.ds(i, L)] = zero

    for de in ds:
        de.wait()

    scale = 1.0 / nopv[pl.ds(0, L)].astype(jnp.float32)
    lane_b = lane + boff
    inv_s = [[plsc.load_gather(invv, [lane_b + (j * 512 + l * 128)]) * scale
              for l in range(3)] for j in range(3)]
    lat_v = [[plsc.load_gather(latv, [lane_b + (k * 512 + i * 128)])
              for i in range(3)] for k in range(3)]

    for o in range(NOP):
        r_v = [[plsc.load_gather(opsv, [lane8 + (k * 512 + l * 128 + o)])
                for l in range(3)] for k in range(3)]
        for j in range(3):
            t1 = [r_v[k][0] * inv_s[j][0] + r_v[k][1] * inv_s[j][1]
                  + r_v[k][2] * inv_s[j][2] for k in range(3)]
            for i in range(3):
                mbuf[pl.ds((o * 9 + j * 3 + i) * L, L)] = (
                    t1[0] * lat_v[0][i] + t1[1] * lat_v[1][i]
                    + t1[2] * lat_v[2][i])

    for de in db:
        de.wait()

    for o in range(NOP):
        m = [[mbuf[pl.ds((o * 9 + j * 3 + i) * L, L)] for i in range(3)]
             for j in range(3)]

        for b0 in range(2):
            f_ref = [rawv.at[pl.ds(b0 * 512 + c * 128, 15616)]
                     for c in range(3)]
            s_ref = smv.at[pl.ds(b0 * 1024 + o * 128, 30848)]

            @pl.loop(0, 128, step=8)
            def _hot(a0):
                alow0 = lane ^ a0
                fidx0 = lane1024 + alow0
                sidx0 = lane2048 + alow0
                for k in range(8):
                    fidx = fidx0 ^ k if k else fidx0
                    sidx = sidx0 ^ k if k else sidx0
                    f0 = plsc.load_gather(f_ref[0], [fidx])
                    f1 = plsc.load_gather(f_ref[1], [fidx])
                    f2 = plsc.load_gather(f_ref[2], [fidx])
                    mv = plsc.load_gather(s_ref, [sidx])
                    widx = (mv << 4) + lane
                    gx = f0 * m[0][0] + f1 * m[1][0] + f2 * m[2][0]
                    gy = f0 * m[0][1] + f1 * m[1][1] + f2 * m[2][1]
                    gz = f0 * m[0][2] + f1 * m[1][2] + f2 * m[2][2]
                    plsc.addupdate_scatter(accx, [widx], gx)
                    plsc.addupdate_scatter(accy, [widx], gy)
                    plsc.addupdate_scatter(accz, [widx], gz)

    for b0 in range(2):
        acc_ref = [accx.at[pl.ds(b0 * 2048, 2048)],
                   accy.at[pl.ds(b0 * 2048, 2048)],
                   accz.at[pl.ds(b0 * 2048, 2048)]]
        w_ref = [rawv.at[pl.ds(b0 * 512 + c * 128, 15616)] for c in range(3)]

        @pl.loop(0, 128, unroll=8)
        def _epi(a0):
            alow = lane ^ a0
            ridx = (alow << 4) + lane
            woff = lane1024 + alow
            plsc.store_scatter(w_ref[0], [woff],
                               plsc.load_gather(acc_ref[0], [ridx]))
            plsc.store_scatter(w_ref[1], [woff],
                               plsc.load_gather(acc_ref[1], [ridx]))
            plsc.store_scatter(w_ref[2], [woff],
                               plsc.load_gather(acc_ref[2], [ridx]))

    pltpu.sync_copy(rawv, out.at[pl.ds(wid * 16384, 16384)])


def kernel(lattices, inv_lattices, forces, batch, num_atoms, general_ops,
           symm_map, num_general_ops):
    B = lattices.shape[0]
    NOP = symm_map.shape[1]
    NA = symm_map.shape[2]
    N = forces.shape[0]

    f_t = (jnp.pad(forces, ((0, 0), (0, 1)))
           .reshape(N // 128, 128, 4).transpose(0, 2, 1).reshape(-1))
    sm_t = (symm_map.reshape(B, NOP, NA // 128, 128)
            .transpose(0, 2, 1, 3).reshape(-1))
    ops_t = (general_ops.reshape(B * NOP // 128, 128, 4, 4)
             .transpose(2, 0, 3, 1).reshape(-1))
    inv_t = (jnp.pad(inv_lattices, ((0, 0), (0, 0), (0, 1)))
             .reshape(B // 128, 128, 3, 4).transpose(2, 0, 3, 1).reshape(-1))
    lat_t = (jnp.pad(lattices, ((0, 0), (0, 0), (0, 1)))
             .reshape(B // 128, 128, 3, 4).transpose(2, 0, 3, 1).reshape(-1))

    mesh = plsc.VectorSubcoreMesh(core_axis_name="c", subcore_axis_name="s",
                                  num_cores=NC, num_subcores=NS)
    run = pl.kernel(
        _sc_body,
        out_type=jax.ShapeDtypeStruct((N * 4,), jnp.float32),
        mesh=mesh,
        compiler_params=pltpu.CompilerParams(needs_layout_passes=False),
        scratch_types=[
            pltpu.VMEM((16384,), jnp.float32),
            pltpu.VMEM((32768,), jnp.int32),
            pltpu.VMEM((1536,), jnp.float32),
            pltpu.VMEM((1536,), jnp.float32),
            pltpu.VMEM((1536,), jnp.float32),
            pltpu.VMEM((L,), jnp.int32),
            pltpu.VMEM((NOP * 9 * L,), jnp.float32),
            pltpu.VMEM((NA * L,), jnp.float32),
            pltpu.VMEM((NA * L,), jnp.float32),
            pltpu.VMEM((NA * L,), jnp.float32),
            pltpu.SemaphoreType.DMA,
            pltpu.SemaphoreType.DMA,
        ],
    )
    out = run(f_t, sm_t, ops_t, inv_t, lat_t, num_general_ops)
    return (out.reshape(N // 128, 4, 128).transpose(0, 2, 1)
            .reshape(N, 4)[:, :3])


# speedup vs baseline: 5.1577x; 1.0086x over previous
import jax
import jax.numpy as jnp
from jax import lax
from jax.experimental import pallas as pl
from jax.experimental.pallas import tpu as pltpu
from jax.experimental.pallas import tpu_sc as plsc

NC = 2
NS = 16
NW = NC * NS
L = 16


def _sc_body(fr, smap, opsr, invr, latr, nopr, out,
             rawv, smva, smvb, opsv, invv, latv, nopv, mbuf,
             accx, accy, accz, semb, sems, semc0, semc1):
    NA = 256
    NOP = 8

    wid = lax.axis_index("c") * NS + lax.axis_index("s")
    blk = wid // 8
    boff = (wid % 8) * 16

    db = pltpu.async_copy(fr.at[pl.ds(wid * 16384, 16384)], rawv, semb)
    sbufs = (smva, smvb)
    csems = (semc0, semc1)
    chunks = [(o, b0) for o in range(NOP) for b0 in range(2)]

    def issue(k):
        o, b0 = chunks[k]
        return pltpu.async_copy(smap.at[wid, :, b0, o, :], sbufs[k % 2],
                                csems[k % 2])

    descs = {0: issue(0), 1: issue(1)}
    ds = []
    for k in range(3):
        ds.append(pltpu.async_copy(
            opsr.at[pl.ds(k * 16384 + wid * 512, 512)],
            opsv.at[pl.ds(k * 512, 512)], sems))
    for j in range(3):
        ds.append(pltpu.async_copy(
            invr.at[pl.ds(j * 2048 + blk * 512, 512)],
            invv.at[pl.ds(j * 512, 512)], sems))
        ds.append(pltpu.async_copy(
            latr.at[pl.ds(j * 2048 + blk * 512, 512)],
            latv.at[pl.ds(j * 512, 512)], sems))
    ds.append(pltpu.async_copy(nopr.at[pl.ds(wid * L, L)], nopv, sems))

    lane = lax.iota(jnp.int32, L)
    lane8 = lane * 8
    lane1024 = lane * 1024
    lane2048 = lane * 2048
    zero = jnp.zeros((L,), jnp.float32)

    @pl.loop(0, NA * L, step=L, unroll=8)
    def _zero(i):
        accx[pl.ds(i, L)] = zero
        accy[pl.ds(i, L)] = zero
        accz[pl.ds(i, L)] = zero

    for de in ds:
        de.wait()

    scale = 1.0 / nopv[pl.ds(0, L)].astype(jnp.float32)
    lane_b = lane + boff
    inv_s = [[plsc.load_gather(invv, [lane_b + (j * 512 + l * 128)]) * scale
              for l in range(3)] for j in range(3)]
    lat_v = [[plsc.load_gather(latv, [lane_b + (k * 512 + i * 128)])
              for i in range(3)] for k in range(3)]

    for o in range(NOP):
        r_v = [[plsc.load_gather(opsv, [lane8 + (k * 512 + l * 128 + o)])
                for l in range(3)] for k in range(3)]
        for j in range(3):
            t1 = [r_v[k][0] * inv_s[j][0] + r_v[k][1] * inv_s[j][1]
                  + r_v[k][2] * inv_s[j][2] for k in range(3)]
            for i in range(3):
                mbuf[pl.ds((o * 9 + j * 3 + i) * L, L)] = (
                    t1[0] * lat_v[0][i] + t1[1] * lat_v[1][i]
                    + t1[2] * lat_v[2][i])

    db.wait()

    for kc, (o, b0) in enumerate(chunks):
        if b0 == 0:
            m = [[mbuf[pl.ds((o * 9 + j * 3 + i) * L, L)] for i in range(3)]
                 for j in range(3)]

        f_ref = [rawv.at[pl.ds(b0 * 512 + c * 128, 15616)] for c in range(3)]
        s_ref = sbufs[kc % 2]
        descs[kc].wait()

        @pl.loop(0, 128, step=8)
        def _hot(a0):
            alow0 = lane ^ a0
            fidx0 = lane1024 + alow0
            for k in range(8):
                alow = alow0 ^ k if k else alow0
                fidx = fidx0 ^ k if k else fidx0
                f0 = plsc.load_gather(f_ref[0], [fidx])
                f1 = plsc.load_gather(f_ref[1], [fidx])
                f2 = plsc.load_gather(f_ref[2], [fidx])
                mv = plsc.load_gather(s_ref, [lane, alow])
                widx = (mv << 4) + lane
                gx = f0 * m[0][0] + f1 * m[1][0] + f2 * m[2][0]
                gy = f0 * m[0][1] + f1 * m[1][1] + f2 * m[2][1]
                gz = f0 * m[0][2] + f1 * m[1][2] + f2 * m[2][2]
                plsc.addupdate_scatter(accx, [widx], gx)
                plsc.addupdate_scatter(accy, [widx], gy)
                plsc.addupdate_scatter(accz, [widx], gz)

        if kc + 2 < len(chunks):
            descs[kc + 2] = issue(kc + 2)

    for b0 in range(2):
        acc_ref = [accx.at[pl.ds(b0 * 2048, 2048)],
                   accy.at[pl.ds(b0 * 2048, 2048)],
                   accz.at[pl.ds(b0 * 2048, 2048)]]
        w_ref = [rawv.at[pl.ds(b0 * 512 + c * 128, 15616)] for c in range(3)]

        @pl.loop(0, 128, unroll=8)
        def _epi(a0):
            alow = lane ^ a0
            ridx = (alow << 4) + lane
            woff = lane1024 + alow
            plsc.store_scatter(w_ref[0], [woff],
                               plsc.load_gather(acc_ref[0], [ridx]))
            plsc.store_scatter(w_ref[1], [woff],
                               plsc.load_gather(acc_ref[1], [ridx]))
            plsc.store_scatter(w_ref[2], [woff],
                               plsc.load_gather(acc_ref[2], [ridx]))

    pltpu.sync_copy(rawv, out.at[pl.ds(wid * 16384, 16384)])


def kernel(lattices, inv_lattices, forces, batch, num_atoms, general_ops,
           symm_map, num_general_ops):
    B = lattices.shape[0]
    NOP = symm_map.shape[1]
    NA = symm_map.shape[2]
    N = forces.shape[0]

    f_t = (jnp.pad(forces, ((0, 0), (0, 1)))
           .reshape(N // 128, 128, 4).transpose(0, 2, 1).reshape(-1))
    sm_t = (symm_map.reshape(B, NOP, NA // 128, 128)
            .transpose(0, 2, 1, 3).reshape(-1))
    ops_t = (general_ops.reshape(B * NOP // 128, 128, 4, 4)
             .transpose(2, 0, 3, 1).reshape(-1))
    inv_t = (jnp.pad(inv_lattices, ((0, 0), (0, 0), (0, 1)))
             .reshape(B // 128, 128, 3, 4).transpose(2, 0, 3, 1).reshape(-1))
    lat_t = (jnp.pad(lattices, ((0, 0), (0, 0), (0, 1)))
             .reshape(B // 128, 128, 3, 4).transpose(2, 0, 3, 1).reshape(-1))

    mesh = plsc.VectorSubcoreMesh(core_axis_name="c", subcore_axis_name="s",
                                  num_cores=NC, num_subcores=NS)
    run = pl.kernel(
        _sc_body,
        out_type=jax.ShapeDtypeStruct((N * 4,), jnp.float32),
        mesh=mesh,
        compiler_params=pltpu.CompilerParams(needs_layout_passes=False),
        scratch_types=[
            pltpu.VMEM((16384,), jnp.float32),
            pltpu.VMEM((L, 128), jnp.int32),
            pltpu.VMEM((L, 128), jnp.int32),
            pltpu.VMEM((1536,), jnp.float32),
            pltpu.VMEM((1536,), jnp.float32),
            pltpu.VMEM((1536,), jnp.float32),
            pltpu.VMEM((L,), jnp.int32),
            pltpu.VMEM((NOP * 9 * L,), jnp.float32),
            pltpu.VMEM((NA * L,), jnp.float32),
            pltpu.VMEM((NA * L,), jnp.float32),
            pltpu.VMEM((NA * L,), jnp.float32),
            pltpu.SemaphoreType.DMA,
            pltpu.SemaphoreType.DMA,
            pltpu.SemaphoreType.DMA,
            pltpu.SemaphoreType.DMA,
        ],
    )
    out = run(f_t, sm_t.reshape(NW, 16, 2, NOP, 128), ops_t, inv_t, lat_t,
              num_general_ops)
    return (out.reshape(N // 128, 4, 128).transpose(0, 2, 1)
            .reshape(N, 4)[:, :3])
